# baseline (speedup 1.0000x reference)
import functools

import jax
import jax.numpy as jnp
from jax import lax
from jax.experimental import pallas as pl
from jax.experimental.pallas import tpu as pltpu
from jax.experimental.pallas import tpu_sc as plsc

_B = 16384
_E = 64
_NCTX = 5
_NC = 2
_NS = 16
_NW = _NC * _NS
_BPW = _B // _NW
_C = 128
_NCHUNK = _BPW // _C
_DPC = _C * _NCTX
_STRIDE = 17


def _sc_dots(tgt_idx, ctx_idx, tgt_table, ctx_table):
    mesh = plsc.VectorSubcoreMesh(core_axis_name="c", subcore_axis_name="s")

    @functools.partial(
        pl.kernel,
        mesh=mesh,
        compiler_params=pltpu.CompilerParams(
            needs_layout_passes=False, use_tc_tiling_on_sc=False),
        out_type=jax.ShapeDtypeStruct((_B * _NCTX,), jnp.float32),
        scratch_types=[
            pltpu.VMEM((_C,), jnp.int32),
            pltpu.VMEM((_NCTX, _C), jnp.int32),
            pltpu.VMEM((_C, _E), jnp.float32),
            pltpu.VMEM((_DPC, _E), jnp.float32),
            pltpu.VMEM((16 * _STRIDE,), jnp.float32),
            pltpu.VMEM((_DPC,), jnp.float32),
            pltpu.SemaphoreType.DMA,
        ],
    )
    def body(tgt_idx_hbm, ctx_idx_hbm, tt_hbm, ct_hbm, out_hbm,
             tgt_idx_v, ctx_idx_v, tgt_rows_v, ctx_rows_v, stage_v, out_v,
             sem):
        wid = lax.axis_index("s") * _NC + lax.axis_index("c")
        iota16 = lax.iota(jnp.int32, 16)
        colbase = [iota16 * _STRIDE + c for c in range(16)]
        rowbase = [iota16 + l * _STRIDE for l in range(16)]

        for chunk in range(_NCHUNK):
            base = wid * _BPW + chunk * _C
            pltpu.sync_copy(tgt_idx_hbm.at[pl.ds(base, _C)], tgt_idx_v)
            for k in range(_NCTX):
                pltpu.sync_copy(
                    ctx_idx_hbm.at[pl.ds(base * _NCTX + k * _C, _C)],
                    ctx_idx_v.at[k])

            copies = [pltpu.async_copy(tt_hbm.at[tgt_idx_v], tgt_rows_v, sem)]
            for k in range(_NCTX):
                copies.append(
                    pltpu.async_copy(ct_hbm.at[ctx_idx_v.at[k]],
                                     ctx_rows_v.at[pl.ds(k * _C, _C)], sem))
            for c in copies:
                c.wait()

            def block_body(blk, _):
                for ii in range(16):
                    i = blk * 16 + ii
                    t = [tgt_rows_v[i, pl.ds(16 * k, 16)] for k in range(4)]
                    for jj in range(_NCTX):
                        d = ii * _NCTX + jj
                        pos = blk * 80 + d
                        p = t[0] * ctx_rows_v[pos, pl.ds(0, 16)]
                        p = p + t[1] * ctx_rows_v[pos, pl.ds(16, 16)]
                        p = p + t[2] * ctx_rows_v[pos, pl.ds(32, 16)]
                        p = p + t[3] * ctx_rows_v[pos, pl.ds(48, 16)]
                        plsc.store_scatter(stage_v, [colbase[d % 16]], p)
                        if d % 16 == 15:
                            acc = plsc.load_gather(stage_v, [rowbase[0]])
                            for l in range(1, 16):
                                acc = acc + plsc.load_gather(
                                    stage_v, [rowbase[l]])
                            g = d // 16
                            out_v[pl.ds(blk * 80 + g * 16, 16)] = acc
                return 0

            lax.fori_loop(0, _C // 16, block_body, 0)
            pltpu.sync_copy(out_v, out_hbm.at[pl.ds(base * _NCTX, _DPC)])

    return body(tgt_idx, ctx_idx, tgt_table, ctx_table)


def kernel(target, context, target_table, context_table):
    tgt_flat = target.reshape(-1).astype(jnp.int32)
    ctx_flat = context.reshape(-1).astype(jnp.int32)
    out_flat = _sc_dots(tgt_flat, ctx_flat, target_table, context_table)
    return out_flat.reshape(_B, _NCTX)


# speedup vs baseline: 1.0009x; 1.0009x over previous
import functools

import jax
import jax.numpy as jnp
from jax import lax
from jax.experimental import pallas as pl
from jax.experimental.pallas import tpu as pltpu
from jax.experimental.pallas import tpu_sc as plsc

_B = 16384
_E = 64
_NCTX = 5
_NC = 2
_NS = 16
_NW = _NC * _NS
_BPW = _B // _NW
_C = 128
_NCHUNK = _BPW // _C
_DPC = _C * _NCTX
_STRIDE = 17


def _sc_dots(tgt_idx, ctx_idx, tgt_table, ctx_table):
    mesh = plsc.VectorSubcoreMesh(core_axis_name="c", subcore_axis_name="s")

    @functools.partial(
        pl.kernel,
        mesh=mesh,
        compiler_params=pltpu.CompilerParams(
            needs_layout_passes=False, use_tc_tiling_on_sc=False),
        out_type=jax.ShapeDtypeStruct((_B * _NCTX,), jnp.float32),
        scratch_types=[
            pltpu.VMEM((_C,), jnp.int32),
            pltpu.VMEM((_NCTX, _C), jnp.int32),
            pltpu.VMEM((_C, _E), jnp.float32),
            pltpu.VMEM((_DPC, _E), jnp.float32),
            pltpu.VMEM((16 * _STRIDE,), jnp.float32),
            pltpu.VMEM((_DPC,), jnp.float32),
            pltpu.SemaphoreType.DMA,
        ],
    )
    def body(tgt_idx_hbm, ctx_idx_hbm, tt_hbm, ct_hbm, out_hbm,
             tgt_idx_v, ctx_idx_v, tgt_rows_v, ctx_rows_v, stage_v, out_v,
             sem):
        wid = lax.axis_index("s") * _NC + lax.axis_index("c")
        iota16 = lax.iota(jnp.int32, 16)
        colbase = [iota16 * _STRIDE + c for c in range(16)]
        rowbase = [iota16 + l * _STRIDE for l in range(16)]

        for chunk in range(_NCHUNK):
            base = wid * _BPW + chunk * _C
            pltpu.sync_copy(tgt_idx_hbm.at[pl.ds(base, _C)], tgt_idx_v)
            for k in range(_NCTX):
                pltpu.sync_copy(
                    ctx_idx_hbm.at[pl.ds(base * _NCTX + k * _C, _C)],
                    ctx_idx_v.at[k])

            copies = [pltpu.async_copy(tt_hbm.at[tgt_idx_v], tgt_rows_v, sem)]
            for k in range(_NCTX):
                copies.append(
                    pltpu.async_copy(ct_hbm.at[ctx_idx_v.at[k]],
                                     ctx_rows_v.at[pl.ds(k * _C, _C)], sem))
            for c in copies:
                c.wait()

            def block_body(blk, _):
                for ii in range(16):
                    i = blk * 16 + ii
                    t = [tgt_rows_v[i, pl.ds(16 * k, 16)] for k in range(4)]
                    for jj in range(_NCTX):
                        d = ii * _NCTX + jj
                        pos = blk * 80 + d
                        p = t[0] * ctx_rows_v[pos, pl.ds(0, 16)]
                        p = p + t[1] * ctx_rows_v[pos, pl.ds(16, 16)]
                        p = p + t[2] * ctx_rows_v[pos, pl.ds(32, 16)]
                        p = p + t[3] * ctx_rows_v[pos, pl.ds(48, 16)]
                        plsc.store_scatter(stage_v, [colbase[d % 16]], p)
                        if d % 16 == 15:
                            acc = plsc.load_gather(stage_v, [rowbase[0]])
                            for l in range(1, 16):
                                acc = acc + plsc.load_gather(
                                    stage_v, [rowbase[l]])
                            g = d // 16
                            out_v[pl.ds(blk * 80 + g * 16, 16)] = acc
                return 0

            lax.fori_loop(0, _C // 16, block_body, 0)
            pltpu.sync_copy(out_v, out_hbm.at[pl.ds(base * _NCTX, _DPC)])

    return body(tgt_idx, ctx_idx, tgt_table, ctx_table)


def kernel(target, context, target_table, context_table):
    tgt_flat = target.reshape(-1).astype(jnp.int32)
    ctx_flat = context.reshape(-1).astype(jnp.int32)
    one = (tgt_flat[0] * 0 + 1).astype(jnp.float32)
    ct_forced = context_table * one
    out_flat = _sc_dots(tgt_flat, ctx_flat, target_table, ct_forced)
    return out_flat.reshape(_B, _NCTX)
